# baseline (device time: 48870 ns/iter reference)
import jax
import jax.numpy as jnp
from jax import lax
from jax.experimental import pallas as pl
from jax.experimental.pallas import tpu as pltpu

M = 1024
HALF = M // 2


def kernel(dy, W):
    dy_bf = dy.astype(jnp.bfloat16)
    w_bf = W.astype(jnp.bfloat16)
    k = dy_bf.shape[1]

    def body(dy_ref, w_ref, out_ref, p_ref, xsend, xrecv, ysend, yrecv,
             xsend_sem, xrecv_sem, ysend_sem, yrecv_sem):
        my_x = lax.axis_index("x")
        my_y = lax.axis_index("y")
        xnbr = (1 - my_x, my_y)
        ynbr = (my_x, 1 - my_y)

        barrier_sem = pltpu.get_barrier_semaphore()
        for nbr in (xnbr, ynbr):
            pl.semaphore_signal(
                barrier_sem, inc=1,
                device_id=nbr, device_id_type=pl.DeviceIdType.MESH,
            )
        pl.semaphore_wait(barrier_sem, 2)

        row0 = my_y * HALF
        p = lax.dot_general(
            dy_ref[pl.ds(row0, HALF), :], w_ref[:, :],
            dimension_numbers=(((1,), (1,)), ((), ())),
            preferred_element_type=jnp.float32,
        )
        p_ref[:, :] = p
        xsend[:, :] = p.astype(jnp.bfloat16)

        rdma_x = pltpu.make_async_remote_copy(
            src_ref=xsend, dst_ref=xrecv,
            send_sem=xsend_sem, recv_sem=xrecv_sem,
            device_id=xnbr, device_id_type=pl.DeviceIdType.MESH,
        )
        rdma_x.start()
        rdma_x.wait()

        red = p_ref[:, :] + xrecv[:, :].astype(jnp.float32)
        out_ref[pl.ds(row0, HALF), :] = red
        ysend[:, :] = red.astype(jnp.bfloat16)

        rdma_y = pltpu.make_async_remote_copy(
            src_ref=ysend, dst_ref=yrecv,
            send_sem=ysend_sem, recv_sem=yrecv_sem,
            device_id=ynbr, device_id_type=pl.DeviceIdType.MESH,
        )
        rdma_y.start()
        rdma_y.wait()

        out_ref[pl.ds((1 - my_y) * HALF, HALF), :] = (
            yrecv[:, :].astype(jnp.float32)
        )

    return pl.pallas_call(
        body,
        out_shape=jax.ShapeDtypeStruct((M, M), jnp.float32),
        in_specs=[
            pl.BlockSpec(memory_space=pltpu.VMEM),
            pl.BlockSpec(memory_space=pltpu.VMEM),
        ],
        out_specs=pl.BlockSpec(memory_space=pltpu.VMEM),
        scratch_shapes=[
            pltpu.VMEM((HALF, M), jnp.float32),
            pltpu.VMEM((HALF, M), jnp.bfloat16),
            pltpu.VMEM((HALF, M), jnp.bfloat16),
            pltpu.VMEM((HALF, M), jnp.bfloat16),
            pltpu.VMEM((HALF, M), jnp.bfloat16),
            pltpu.SemaphoreType.DMA,
            pltpu.SemaphoreType.DMA,
            pltpu.SemaphoreType.DMA,
            pltpu.SemaphoreType.DMA,
        ],
        compiler_params=pltpu.CompilerParams(collective_id=0),
    )(dy_bf, w_bf)
